# baseline (speedup 1.0000x reference)
import functools

import jax
import jax.numpy as jnp
from jax import lax
from jax.experimental import pallas as pl
from jax.experimental.pallas import tpu as pltpu
from jax.experimental.pallas import tpu_sc as plsc

NUM_BUCKETS = 100000
DIM = 32
MODEL_DIM = 1024
BATCH = 4
SEQ = 4096
TOK = BATCH * SEQ
NK = 4
CONCAT = NK * DIM

NC = 2
NS = 16
NW = NC * NS
TPW = TOK // NW
ICH = 128
NCH = TPW // ICH


def _mod_buckets(x):
    q = (x.astype(jnp.float32) * jnp.float32(1e-5)).astype(jnp.int32)
    r = x - q * NUM_BUCKETS
    r = jnp.where(r < 0, r + NUM_BUCKETS, r)
    r = jnp.where(r >= NUM_BUCKETS, r - NUM_BUCKETS, r)
    return r


def _perm_rows(i):
    return (i & jnp.int32(-512)) | ((i & 127) << 2) | ((i >> 7) & 3)


def _sc_body(ids_hbm, embed_hbm, out_hbm, buf_v, idx0_v, idx1_v, idx2_v,
             idx3_v, rows_v, sem):
    wid = lax.axis_index("s") * NC + lax.axis_index("c")
    base = wid * TPW

    pltpu.sync_copy(ids_hbm.at[pl.ds(base, TPW)], buf_v.at[pl.ds(16, TPW)])
    row_start = wid % (SEQ // TPW) == 0

    @pl.when(row_start)
    def _():
        buf_v[pl.ds(0, 16)] = jnp.zeros((16,), jnp.int32)

    @pl.when(jnp.logical_not(row_start))
    def _():
        pltpu.sync_copy(ids_hbm.at[pl.ds(base - 16, 16)], buf_v.at[pl.ds(0, 16)])

    lanes = lax.iota(jnp.int32, 16)
    idx_refs = (idx0_v, idx1_v, idx2_v, idx3_v)
    for i in range(NCH):
        for j in range(ICH // 16):
            off = i * ICH + j * 16
            c = buf_v[pl.ds(16 + off, 16)]
            p = buf_v[pl.ds(15 + off, 16)]
            q = buf_v[pl.ds(14 + off, 16)]
            h0 = _mod_buckets(p * 1009 + c)
            h1 = _mod_buckets((p * 2719 + 314159) ^ (c * 3137)) + NUM_BUCKETS
            h2 = _mod_buckets((q * 36313) ^ (p * 27191) ^ (c * 4903)) + 2 * NUM_BUCKETS
            h3 = _mod_buckets((q * 7919) ^ (p * 4391) ^ (c * 6151)) + 3 * NUM_BUCKETS
            for k, h in enumerate((h0, h1, h2, h3)):
                idx_refs[k][i, pl.ds(j * 16, 16)] = _perm_rows(h)

    for k in range(NK):
        copies = [
            pltpu.async_copy(embed_hbm.at[idx_refs[k].at[jnp.int32(i)]],
                             rows_v.at[pl.ds(i * ICH, ICH)], sem)
            for i in range(NCH)
        ]
        for c_ in copies:
            c_.wait()
        pltpu.sync_copy(rows_v,
                        out_hbm.at[pl.ds(base, TPW), pl.ds(k * DIM, DIM)])


def _sc_gather(ids, embed):
    f = functools.partial(
        pl.kernel,
        out_type=jax.ShapeDtypeStruct((TOK, CONCAT), jnp.float32),
        scratch_types=[
            pltpu.VMEM((TPW + 16,), jnp.int32),
            pltpu.VMEM((NCH, ICH), jnp.int32),
            pltpu.VMEM((NCH, ICH), jnp.int32),
            pltpu.VMEM((NCH, ICH), jnp.int32),
            pltpu.VMEM((NCH, ICH), jnp.int32),
            pltpu.VMEM((TPW, DIM), jnp.float32),
            pltpu.SemaphoreType.DMA,
        ],
        mesh=plsc.VectorSubcoreMesh(core_axis_name="c", subcore_axis_name="s"),
        compiler_params=pltpu.CompilerParams(use_tc_tiling_on_sc=False),
    )(_sc_body)
    return f(ids, embed)


TM = 512


def _i32(v):
    return jnp.int32(v)


def _mm_body(x_ref, w_ref, g_ref, o_ref):
    g = jax.nn.sigmoid(g_ref[...])
    acc = lax.dot_general(x_ref[...], w_ref[...], (((1,), (1,)), ((), ())),
                          preferred_element_type=jnp.float32)
    o_ref[...] = acc * g


def _project(gathered, w, gate2d):
    return pl.pallas_call(
        _mm_body,
        grid=(TOK // TM,),
        in_specs=[
            pl.BlockSpec((TM, CONCAT), lambda m: (m, _i32(0))),
            pl.BlockSpec((MODEL_DIM, CONCAT), lambda m: (_i32(0), _i32(0))),
            pl.BlockSpec((1, MODEL_DIM), lambda m: (_i32(0), _i32(0))),
        ],
        out_specs=pl.BlockSpec((TM, MODEL_DIM), lambda m: (m, _i32(0))),
        out_shape=jax.ShapeDtypeStruct((TOK, MODEL_DIM), jnp.float32),
        compiler_params=pltpu.CompilerParams(
            dimension_semantics=("parallel",)),
    )(gathered, w, gate2d)


TRGRID = 782
ROWS_P = TRGRID * 512


def _tr_body(e_ref, o_ref):
    x = e_ref[...]
    u = jnp.concatenate(
        [x[:, 0:128], x[:, 128:256], x[:, 256:384], x[:, 384:512]],
        axis=0)
    o_ref[...] = jnp.swapaxes(u, 0, 1)


def _linearize(embT):
    return pl.pallas_call(
        _tr_body,
        grid=(TRGRID,),
        in_specs=[pl.BlockSpec((DIM, 512), lambda i: (_i32(0), i))],
        out_specs=pl.BlockSpec((128, 128), lambda i: (i, _i32(0))),
        out_shape=jax.ShapeDtypeStruct((ROWS_P * DIM // 128, 128),
                                       jnp.float32),
        compiler_params=pltpu.CompilerParams(
            dimension_semantics=("arbitrary",)),
    )(embT)


def kernel(input_ids, embed, W, ngram_gate):
    ids = input_ids.astype(jnp.int32).reshape(TOK)
    lin2d = _linearize(embed.T)
    table = lin2d.reshape(-1).reshape(ROWS_P, DIM)
    gathered = _sc_gather(ids, table)
    gate2d = ngram_gate.reshape(1, MODEL_DIM)
    out = _project(gathered, W, gate2d)
    return out.reshape(BATCH, SEQ, MODEL_DIM)


# speedup vs baseline: 3.4063x; 3.4063x over previous
import functools

import jax
import jax.numpy as jnp
from jax import lax
from jax.experimental import pallas as pl
from jax.experimental.pallas import tpu as pltpu
from jax.experimental.pallas import tpu_sc as plsc

NUM_BUCKETS = 100000
DIM = 32
MODEL_DIM = 1024
BATCH = 4
SEQ = 4096
TOK = BATCH * SEQ
NK = 4
CONCAT = NK * DIM

NC = 2
NS = 16
NW = NC * NS
TPW = TOK // NW
ICH = 128
NCH = TPW // ICH


def _mod_buckets(x):
    q = (x.astype(jnp.float32) * jnp.float32(1e-5)).astype(jnp.int32)
    r = x - q * NUM_BUCKETS
    r = jnp.where(r < 0, r + NUM_BUCKETS, r)
    r = jnp.where(r >= NUM_BUCKETS, r - NUM_BUCKETS, r)
    return r


def _perm_rows(i):
    return (i & jnp.int32(-512)) | ((i & 127) << 2) | ((i >> 7) & 3)


def _sc_body(ids_hbm, embed_hbm, out_hbm, buf_v, idx0_v, idx1_v, idx2_v,
             idx3_v, rows_v, sem):
    wid = lax.axis_index("s") * NC + lax.axis_index("c")
    base = wid * TPW

    pltpu.sync_copy(ids_hbm.at[pl.ds(base, TPW)], buf_v.at[pl.ds(16, TPW)])
    row_start = wid % (SEQ // TPW) == 0

    @pl.when(row_start)
    def _():
        buf_v[pl.ds(0, 16)] = jnp.zeros((16,), jnp.int32)

    @pl.when(jnp.logical_not(row_start))
    def _():
        pltpu.sync_copy(ids_hbm.at[pl.ds(base - 16, 16)], buf_v.at[pl.ds(0, 16)])

    lanes = lax.iota(jnp.int32, 16)
    idx_refs = (idx0_v, idx1_v, idx2_v, idx3_v)
    for i in range(NCH):
        for j in range(ICH // 16):
            off = i * ICH + j * 16
            c = buf_v[pl.ds(16 + off, 16)]
            p = buf_v[pl.ds(15 + off, 16)]
            q = buf_v[pl.ds(14 + off, 16)]
            h0 = _mod_buckets(p * 1009 + c)
            h1 = _mod_buckets((p * 2719 + 314159) ^ (c * 3137)) + NUM_BUCKETS
            h2 = _mod_buckets((q * 36313) ^ (p * 27191) ^ (c * 4903)) + 2 * NUM_BUCKETS
            h3 = _mod_buckets((q * 7919) ^ (p * 4391) ^ (c * 6151)) + 3 * NUM_BUCKETS
            for k, h in enumerate((h0, h1, h2, h3)):
                idx_refs[k][i, pl.ds(j * 16, 16)] = _perm_rows(h)

    for k in range(NK):
        copies = [
            pltpu.async_copy(embed_hbm.at[idx_refs[k].at[jnp.int32(i)]],
                             rows_v.at[pl.ds(i * ICH, ICH)], sem)
            for i in range(NCH)
        ]
        for c_ in copies:
            c_.wait()
        pltpu.sync_copy(rows_v,
                        out_hbm.at[pl.ds(base, TPW), pl.ds(k * DIM, DIM)])


def _sc_gather(ids, embed):
    f = functools.partial(
        pl.kernel,
        out_type=jax.ShapeDtypeStruct((TOK, CONCAT), jnp.float32),
        scratch_types=[
            pltpu.VMEM((TPW + 16,), jnp.int32),
            pltpu.VMEM((NCH, ICH), jnp.int32),
            pltpu.VMEM((NCH, ICH), jnp.int32),
            pltpu.VMEM((NCH, ICH), jnp.int32),
            pltpu.VMEM((NCH, ICH), jnp.int32),
            pltpu.VMEM((TPW, DIM), jnp.float32),
            pltpu.SemaphoreType.DMA,
        ],
        mesh=plsc.VectorSubcoreMesh(core_axis_name="c", subcore_axis_name="s"),
        compiler_params=pltpu.CompilerParams(use_tc_tiling_on_sc=False),
    )(_sc_body)
    return f(ids, embed)


TM = 512


def _i32(v):
    return jnp.int32(v)


def _mm_body(x_ref, w_ref, g_ref, o_ref):
    g = jax.nn.sigmoid(g_ref[...])
    acc = lax.dot_general(x_ref[...], w_ref[...], (((1,), (1,)), ((), ())),
                          preferred_element_type=jnp.float32)
    o_ref[...] = acc * g


def _project(gathered, w, gate2d):
    return pl.pallas_call(
        _mm_body,
        grid=(TOK // TM,),
        in_specs=[
            pl.BlockSpec((TM, CONCAT), lambda m: (m, _i32(0))),
            pl.BlockSpec((MODEL_DIM, CONCAT), lambda m: (_i32(0), _i32(0))),
            pl.BlockSpec((1, MODEL_DIM), lambda m: (_i32(0), _i32(0))),
        ],
        out_specs=pl.BlockSpec((TM, MODEL_DIM), lambda m: (m, _i32(0))),
        out_shape=jax.ShapeDtypeStruct((TOK, MODEL_DIM), jnp.float32),
        compiler_params=pltpu.CompilerParams(
            dimension_semantics=("parallel",)),
    )(gathered, w, gate2d)


TRCH = 8
TRGRID = 98
ROWS_P = TRGRID * 512 * TRCH


def _tr_body(e_ref, o_ref):
    x = e_ref[...]
    outs = []
    for t in range(TRCH):
        xc = x[:, t * 512:(t + 1) * 512]
        u = jnp.concatenate(
            [xc[:, 0:128], xc[:, 128:256], xc[:, 256:384], xc[:, 384:512]],
            axis=0)
        outs.append(jnp.swapaxes(u, 0, 1))
    o_ref[...] = jnp.concatenate(outs, axis=0)


def _linearize(embT):
    return pl.pallas_call(
        _tr_body,
        grid=(TRGRID,),
        in_specs=[pl.BlockSpec((DIM, 512 * TRCH), lambda i: (_i32(0), i))],
        out_specs=pl.BlockSpec((128 * TRCH, 128), lambda i: (i, _i32(0))),
        out_shape=jax.ShapeDtypeStruct((ROWS_P * DIM // 128, 128),
                                       jnp.float32),
        compiler_params=pltpu.CompilerParams(
            dimension_semantics=("arbitrary",)),
    )(embT)


def kernel(input_ids, embed, W, ngram_gate):
    ids = input_ids.astype(jnp.int32).reshape(TOK)
    lin2d = _linearize(embed.T)
    table = lin2d.reshape(-1).reshape(ROWS_P, DIM)
    gathered = _sc_gather(ids, table)
    gate2d = ngram_gate.reshape(1, MODEL_DIM)
    out = _project(gathered, W, gate2d)
    return out.reshape(BATCH, SEQ, MODEL_DIM)


# speedup vs baseline: 3.9990x; 1.1740x over previous
import functools

import jax
import jax.numpy as jnp
from jax import lax
from jax.experimental import pallas as pl
from jax.experimental.pallas import tpu as pltpu
from jax.experimental.pallas import tpu_sc as plsc

NUM_BUCKETS = 100000
DIM = 32
MODEL_DIM = 1024
BATCH = 4
SEQ = 4096
TOK = BATCH * SEQ
NK = 4
CONCAT = NK * DIM

NC = 2
NS = 16
NW = NC * NS
TPW = TOK // NW
ICH = 128
NCH = TPW // ICH


def _mod_buckets(x):
    q = (x.astype(jnp.float32) * jnp.float32(1e-5)).astype(jnp.int32)
    r = x - q * NUM_BUCKETS
    r = jnp.where(r < 0, r + NUM_BUCKETS, r)
    r = jnp.where(r >= NUM_BUCKETS, r - NUM_BUCKETS, r)
    return r


def _perm_rows(i):
    return (i & jnp.int32(-512)) | ((i & 127) << 2) | ((i >> 7) & 3)


def _sc_body(ids_hbm, embed_hbm, out_hbm, buf_v, idx0_v, idx1_v, idx2_v,
             idx3_v, rows_v, sem):
    wid = lax.axis_index("s") * NC + lax.axis_index("c")
    base = wid * TPW

    pltpu.sync_copy(ids_hbm.at[pl.ds(base, TPW)], buf_v.at[pl.ds(16, TPW)])
    row_start = wid % (SEQ // TPW) == 0

    @pl.when(row_start)
    def _():
        buf_v[pl.ds(0, 16)] = jnp.zeros((16,), jnp.int32)

    @pl.when(jnp.logical_not(row_start))
    def _():
        pltpu.sync_copy(ids_hbm.at[pl.ds(base - 16, 16)], buf_v.at[pl.ds(0, 16)])

    lanes = lax.iota(jnp.int32, 16)
    idx_refs = (idx0_v, idx1_v, idx2_v, idx3_v)
    for i in range(NCH):
        for j in range(ICH // 16):
            off = i * ICH + j * 16
            c = buf_v[pl.ds(16 + off, 16)]
            p = buf_v[pl.ds(15 + off, 16)]
            q = buf_v[pl.ds(14 + off, 16)]
            h0 = _mod_buckets(p * 1009 + c)
            h1 = _mod_buckets((p * 2719 + 314159) ^ (c * 3137)) + NUM_BUCKETS
            h2 = _mod_buckets((q * 36313) ^ (p * 27191) ^ (c * 4903)) + 2 * NUM_BUCKETS
            h3 = _mod_buckets((q * 7919) ^ (p * 4391) ^ (c * 6151)) + 3 * NUM_BUCKETS
            for k, h in enumerate((h0, h1, h2, h3)):
                idx_refs[k][i, pl.ds(j * 16, 16)] = _perm_rows(h)

    for k in range(NK):
        copies = [
            pltpu.async_copy(embed_hbm.at[idx_refs[k].at[jnp.int32(i)]],
                             rows_v.at[pl.ds(i * ICH, ICH)], sem)
            for i in range(NCH)
        ]
        for c_ in copies:
            c_.wait()
        pltpu.sync_copy(rows_v,
                        out_hbm.at[pl.ds(base, TPW), pl.ds(k * DIM, DIM)])


def _sc_gather(ids, embed):
    f = functools.partial(
        pl.kernel,
        out_type=jax.ShapeDtypeStruct((TOK, CONCAT), jnp.float32),
        scratch_types=[
            pltpu.VMEM((TPW + 16,), jnp.int32),
            pltpu.VMEM((NCH, ICH), jnp.int32),
            pltpu.VMEM((NCH, ICH), jnp.int32),
            pltpu.VMEM((NCH, ICH), jnp.int32),
            pltpu.VMEM((NCH, ICH), jnp.int32),
            pltpu.VMEM((TPW, DIM), jnp.float32),
            pltpu.SemaphoreType.DMA,
        ],
        mesh=plsc.VectorSubcoreMesh(core_axis_name="c", subcore_axis_name="s"),
        compiler_params=pltpu.CompilerParams(use_tc_tiling_on_sc=False),
    )(_sc_body)
    return f(ids, embed)


TM = 512


def _i32(v):
    return jnp.int32(v)


def _mm_body(x_ref, w_ref, g_ref, o_ref):
    g = jax.nn.sigmoid(g_ref[...])
    acc = lax.dot_general(x_ref[...], w_ref[...], (((1,), (1,)), ((), ())),
                          preferred_element_type=jnp.float32)
    o_ref[...] = acc * g


def _project(gathered, w, gate2d):
    return pl.pallas_call(
        _mm_body,
        grid=(TOK // TM,),
        in_specs=[
            pl.BlockSpec((TM, CONCAT), lambda m: (m, _i32(0))),
            pl.BlockSpec((MODEL_DIM, CONCAT), lambda m: (_i32(0), _i32(0))),
            pl.BlockSpec((1, MODEL_DIM), lambda m: (_i32(0), _i32(0))),
        ],
        out_specs=pl.BlockSpec((TM, MODEL_DIM), lambda m: (m, _i32(0))),
        out_shape=jax.ShapeDtypeStruct((TOK, MODEL_DIM), jnp.float32),
        compiler_params=pltpu.CompilerParams(
            dimension_semantics=("parallel",)),
    )(gathered, w, gate2d)


TRCH = 16
TRGRID = 49
ROWS_P = TRGRID * 512 * TRCH


def _tr_body(e_ref, o_ref):
    x = e_ref[...]
    outs = []
    for t in range(TRCH):
        xc = x[:, t * 512:(t + 1) * 512]
        u = jnp.concatenate(
            [xc[:, 0:128], xc[:, 128:256], xc[:, 256:384], xc[:, 384:512]],
            axis=0)
        outs.append(jnp.swapaxes(u, 0, 1))
    o_ref[...] = jnp.concatenate(outs, axis=0)


def _linearize(embT):
    return pl.pallas_call(
        _tr_body,
        grid=(TRGRID,),
        in_specs=[pl.BlockSpec((DIM, 512 * TRCH), lambda i: (_i32(0), i))],
        out_specs=pl.BlockSpec((128 * TRCH, 128), lambda i: (i, _i32(0))),
        out_shape=jax.ShapeDtypeStruct((ROWS_P * DIM // 128, 128),
                                       jnp.float32),
        compiler_params=pltpu.CompilerParams(
            dimension_semantics=("arbitrary",)),
    )(embT)


def kernel(input_ids, embed, W, ngram_gate):
    ids = input_ids.astype(jnp.int32).reshape(TOK)
    lin2d = _linearize(embed.T)
    table = lin2d.reshape(-1).reshape(ROWS_P, DIM)
    gathered = _sc_gather(ids, table)
    gate2d = ngram_gate.reshape(1, MODEL_DIM)
    out = _project(gathered, W, gate2d)
    return out.reshape(BATCH, SEQ, MODEL_DIM)


# speedup vs baseline: 4.5811x; 1.1455x over previous
import functools

import jax
import jax.numpy as jnp
from jax import lax
from jax.experimental import pallas as pl
from jax.experimental.pallas import tpu as pltpu
from jax.experimental.pallas import tpu_sc as plsc

NUM_BUCKETS = 100000
DIM = 32
MODEL_DIM = 1024
BATCH = 4
SEQ = 4096
TOK = BATCH * SEQ
NK = 4
CONCAT = NK * DIM

NC = 2
NS = 16
NW = NC * NS
TPW = TOK // NW
ICH = 128
NCH = TPW // ICH


def _mod_buckets(x):
    q = (x.astype(jnp.float32) * jnp.float32(1e-5)).astype(jnp.int32)
    r = x - q * NUM_BUCKETS
    r = jnp.where(r < 0, r + NUM_BUCKETS, r)
    r = jnp.where(r >= NUM_BUCKETS, r - NUM_BUCKETS, r)
    return r


def _perm_rows(i):
    return (i & jnp.int32(-512)) | ((i & 127) << 2) | ((i >> 7) & 3)


def _sc_body(ids_hbm, embed_hbm, out_hbm, buf_v, idx0_v, idx1_v, idx2_v,
             idx3_v, rows_v, sem):
    wid = lax.axis_index("s") * NC + lax.axis_index("c")
    base = wid * TPW

    pltpu.sync_copy(ids_hbm.at[pl.ds(base, TPW)], buf_v.at[pl.ds(16, TPW)])
    row_start = wid % (SEQ // TPW) == 0

    @pl.when(row_start)
    def _():
        buf_v[pl.ds(0, 16)] = jnp.zeros((16,), jnp.int32)

    @pl.when(jnp.logical_not(row_start))
    def _():
        pltpu.sync_copy(ids_hbm.at[pl.ds(base - 16, 16)], buf_v.at[pl.ds(0, 16)])

    lanes = lax.iota(jnp.int32, 16)
    idx_refs = (idx0_v, idx1_v, idx2_v, idx3_v)
    for i in range(NCH):
        for j in range(ICH // 16):
            off = i * ICH + j * 16
            c = buf_v[pl.ds(16 + off, 16)]
            p = buf_v[pl.ds(15 + off, 16)]
            q = buf_v[pl.ds(14 + off, 16)]
            h0 = _mod_buckets(p * 1009 + c)
            h1 = _mod_buckets((p * 2719 + 314159) ^ (c * 3137)) + NUM_BUCKETS
            h2 = _mod_buckets((q * 36313) ^ (p * 27191) ^ (c * 4903)) + 2 * NUM_BUCKETS
            h3 = _mod_buckets((q * 7919) ^ (p * 4391) ^ (c * 6151)) + 3 * NUM_BUCKETS
            for k, h in enumerate((h0, h1, h2, h3)):
                idx_refs[k][i, pl.ds(j * 16, 16)] = _perm_rows(h)

    for k in range(NK):
        copies = [
            pltpu.async_copy(embed_hbm.at[idx_refs[k].at[jnp.int32(i)]],
                             rows_v.at[pl.ds(i * ICH, ICH)], sem)
            for i in range(NCH)
        ]
        for c_ in copies:
            c_.wait()
        pltpu.sync_copy(rows_v,
                        out_hbm.at[pl.ds(base, TPW), pl.ds(k * DIM, DIM)])


def _sc_gather(ids, embed):
    f = functools.partial(
        pl.kernel,
        out_type=jax.ShapeDtypeStruct((TOK, CONCAT), jnp.float32),
        scratch_types=[
            pltpu.VMEM((TPW + 16,), jnp.int32),
            pltpu.VMEM((NCH, ICH), jnp.int32),
            pltpu.VMEM((NCH, ICH), jnp.int32),
            pltpu.VMEM((NCH, ICH), jnp.int32),
            pltpu.VMEM((NCH, ICH), jnp.int32),
            pltpu.VMEM((TPW, DIM), jnp.float32),
            pltpu.SemaphoreType.DMA,
        ],
        mesh=plsc.VectorSubcoreMesh(core_axis_name="c", subcore_axis_name="s"),
        compiler_params=pltpu.CompilerParams(use_tc_tiling_on_sc=False),
    )(_sc_body)
    return f(ids, embed)


TM = 512


def _i32(v):
    return jnp.int32(v)


def _mm_body(x_ref, w_ref, g_ref, o_ref):
    g = jax.nn.sigmoid(g_ref[...])
    acc = lax.dot_general(x_ref[...], w_ref[...], (((1,), (1,)), ((), ())),
                          preferred_element_type=jnp.float32)
    o_ref[...] = acc * g


def _project(gathered, w, gate2d):
    return pl.pallas_call(
        _mm_body,
        grid=(TOK // TM,),
        in_specs=[
            pl.BlockSpec((TM, CONCAT), lambda m: (m, _i32(0))),
            pl.BlockSpec((MODEL_DIM, CONCAT), lambda m: (_i32(0), _i32(0))),
            pl.BlockSpec((1, MODEL_DIM), lambda m: (_i32(0), _i32(0))),
        ],
        out_specs=pl.BlockSpec((TM, MODEL_DIM), lambda m: (m, _i32(0))),
        out_shape=jax.ShapeDtypeStruct((TOK, MODEL_DIM), jnp.float32),
        compiler_params=pltpu.CompilerParams(
            dimension_semantics=("parallel",)),
    )(gathered, w, gate2d)


TRCH = 32
TRGRID = 25
ROWS_P = TRGRID * 512 * TRCH


def _tr_body(e_ref, o_ref):
    x = e_ref[...]
    outs = []
    for t in range(TRCH):
        xc = x[:, t * 512:(t + 1) * 512]
        u = jnp.concatenate(
            [xc[:, 0:128], xc[:, 128:256], xc[:, 256:384], xc[:, 384:512]],
            axis=0)
        outs.append(jnp.swapaxes(u, 0, 1))
    o_ref[...] = jnp.concatenate(outs, axis=0)


def _linearize(embT):
    return pl.pallas_call(
        _tr_body,
        grid=(TRGRID,),
        in_specs=[pl.BlockSpec((DIM, 512 * TRCH), lambda i: (_i32(0), i))],
        out_specs=pl.BlockSpec((128 * TRCH, 128), lambda i: (i, _i32(0))),
        out_shape=jax.ShapeDtypeStruct((ROWS_P * DIM // 128, 128),
                                       jnp.float32),
        compiler_params=pltpu.CompilerParams(
            dimension_semantics=("arbitrary",)),
    )(embT)


def kernel(input_ids, embed, W, ngram_gate):
    ids = input_ids.astype(jnp.int32).reshape(TOK)
    lin2d = _linearize(embed.T)
    table = lin2d.reshape(-1).reshape(ROWS_P, DIM)
    gathered = _sc_gather(ids, table)
    gate2d = ngram_gate.reshape(1, MODEL_DIM)
    out = _project(gathered, W, gate2d)
    return out.reshape(BATCH, SEQ, MODEL_DIM)
